# baseline (speedup 1.0000x reference)
import functools

import jax
import jax.numpy as jnp
from jax.experimental import pallas as pl
from jax.experimental.pallas import tpu as pltpu
from jax.experimental.pallas import tpu_sc as plsc

K_BLOCK = 2048
SC_WINDOW = 128


def _argmin_dist_kernel(x_ref, cb_ref, idx_ref, rm_ref, ri_ref,
                        xm2_ref, xb_ref, *, n_steps):
    k = pl.program_id(0)

    @pl.when(k == 0)
    def _init():
        rm_ref[...] = jnp.full(rm_ref.shape, jnp.inf, dtype=rm_ref.dtype)
        x = x_ref[...]
        xm2_ref[...] = x * (-2.0)
        xsq = jnp.sum(jnp.square(x), axis=1, keepdims=True)
        xb_ref[...] = jnp.broadcast_to(xsq, xb_ref.shape)

    cb = cb_ref[...]
    c_sq = jnp.sum(jnp.square(cb), axis=1, keepdims=True)
    c_sq_row = jax.lax.transpose(c_sq, (1, 0))

    cross2 = jax.lax.dot_general(
        xm2_ref[...], cb,
        dimension_numbers=(((1,), (1,)), ((), ())),
        preferred_element_type=jnp.float32,
    )
    dist = (xb_ref[...] + cross2) + c_sq_row

    ent = [(dist[:, g * 128:(g + 1) * 128], jnp.int32(g * 128))
           for g in range(K_BLOCK // 128)]
    vals = [e[0] for e in ent]
    offs = [e[1] for e in ent]
    while len(vals) > 1:
        nv, no = [], []
        for a in range(0, len(vals), 2):
            lo, hi = vals[a], vals[a + 1]
            h = hi < lo
            nv.append(jnp.minimum(lo, hi))
            no.append(jnp.where(h, offs[a + 1], offs[a]))
        vals, offs = nv, no
    m, cv = vals[0], offs[0]
    iota_k = (jax.lax.broadcasted_iota(jnp.int32, m.shape, 1)
              + k * K_BLOCK)
    cand = cv + iota_k

    old = rm_ref[...]
    upd = m < old
    rm_ref[...] = jnp.minimum(m, old)
    ri_ref[...] = jnp.where(upd, cand, ri_ref[...])

    @pl.when(k == n_steps - 1)
    def _finalize():
        rm = rm_ref[...]
        ri = ri_ref[...]
        rowmin = jnp.min(rm, axis=1, keepdims=True)
        cand_f = jnp.where(rm == rowmin, ri, jnp.int32(2**30))
        idx_col = jnp.min(cand_f, axis=1, keepdims=True)
        idx_ref[...] = jax.lax.transpose(idx_col, (1, 0))


def _tc_argmin(x, codebook):
    batch, dim = x.shape
    n_codes = codebook.shape[0]
    n_steps = n_codes // K_BLOCK
    grid_kernel = functools.partial(_argmin_dist_kernel, n_steps=n_steps)
    return pl.pallas_call(
        grid_kernel,
        grid=(n_steps,),
        in_specs=[
            pl.BlockSpec((batch, dim), lambda k: (0, 0)),
            pl.BlockSpec((K_BLOCK, dim), lambda k: (k, 0)),
        ],
        out_specs=pl.BlockSpec((1, batch), lambda k: (0, 0)),
        out_shape=jax.ShapeDtypeStruct((1, batch), jnp.int32),
        scratch_shapes=[
            pltpu.VMEM((batch, 128), jnp.float32),
            pltpu.VMEM((batch, 128), jnp.int32),
            pltpu.VMEM((batch, dim), jnp.float32),
            pltpu.VMEM((batch, K_BLOCK), jnp.float32),
        ],
        compiler_params=pltpu.CompilerParams(
            dimension_semantics=("arbitrary",),
        ),
    )(x, codebook)


def _sc_gather(codebook, idx_row):
    n_codes, dim = codebook.shape
    batch = idx_row.shape[1]
    mesh = plsc.VectorSubcoreMesh(core_axis_name="core",
                                  subcore_axis_name="subcore")

    @functools.partial(
        pl.kernel,
        out_type=jax.ShapeDtypeStruct((batch, dim), jnp.float32),
        mesh=mesh,
    )
    def sc_kernel(cb_hbm, i_hbm, o_hbm):
        def body(i_vmem, o_vmem):
            pltpu.sync_copy(cb_hbm.at[i_vmem.at[0]], o_vmem)

        pltpu.emit_pipeline(
            body,
            grid=(batch // SC_WINDOW,),
            in_specs=[
                pl.BlockSpec((1, SC_WINDOW), lambda i: (0, i)),
            ],
            out_specs=[pl.BlockSpec((SC_WINDOW, dim), lambda i: (i, 0))],
            core_axis_name=("core", "subcore"),
            dimension_semantics=(pltpu.PARALLEL,),
        )(i_hbm, o_hbm)

    return sc_kernel(codebook, idx_row)


def kernel(previous_residual, codebook_embeddings):
    batch = previous_residual.shape[0]
    idx_row = _tc_argmin(previous_residual, codebook_embeddings)
    gathered = _sc_gather(codebook_embeddings, idx_row)
    next_residual = previous_residual - gathered
    return (idx_row.reshape(batch), next_residual)


# speedup vs baseline: 1.0407x; 1.0407x over previous
import functools

import jax
import jax.numpy as jnp
from jax.experimental import pallas as pl
from jax.experimental.pallas import tpu as pltpu
from jax.experimental.pallas import tpu_sc as plsc

K_BLOCK = 1024
SC_WINDOW = 128


def _argmin_dist_kernel(x_ref, cb_ref, idx_ref, rm_ref, ri_ref,
                        xm2_ref, xb_ref, *, n_steps):
    k = pl.program_id(0)

    @pl.when(k == 0)
    def _init():
        rm_ref[...] = jnp.full(rm_ref.shape, jnp.inf, dtype=rm_ref.dtype)
        x = x_ref[...]
        xm2_ref[...] = x * (-2.0)
        xsq = jnp.sum(jnp.square(x), axis=1, keepdims=True)
        xb_ref[...] = jnp.broadcast_to(xsq, xb_ref.shape)

    cb = cb_ref[...]
    c_sq = jnp.sum(jnp.square(cb), axis=1, keepdims=True)
    c_sq_row = jax.lax.transpose(c_sq, (1, 0))

    cross2 = jax.lax.dot_general(
        xm2_ref[...], cb,
        dimension_numbers=(((1,), (1,)), ((), ())),
        preferred_element_type=jnp.float32,
    )
    dist = (xb_ref[...] + cross2) + c_sq_row

    ent = [(dist[:, g * 128:(g + 1) * 128], jnp.float32(g * 128))
           for g in range(K_BLOCK // 128)]
    vals = [e[0] for e in ent]
    offs = [e[1] for e in ent]
    while len(vals) > 1:
        nv, no = [], []
        for a in range(0, len(vals), 2):
            lo, hi = vals[a], vals[a + 1]
            h = hi < lo
            nv.append(jnp.minimum(lo, hi))
            no.append(jnp.where(h, offs[a + 1], offs[a]))
        vals, offs = nv, no
    m, cv = vals[0], offs[0]
    iota_k = (jax.lax.broadcasted_iota(jnp.int32, m.shape, 1)
              .astype(jnp.float32) + (k * K_BLOCK).astype(jnp.float32))
    cand = cv + iota_k

    old = rm_ref[...]
    upd = m < old
    rm_ref[...] = jnp.minimum(m, old)
    ri_ref[...] = jnp.where(upd, cand, ri_ref[...])

    @pl.when(k == n_steps - 1)
    def _finalize():
        rm = rm_ref[...]
        ri = ri_ref[...]
        rowmin = jnp.min(rm, axis=1, keepdims=True)
        cand_f = jnp.where(rm == rowmin, ri, jnp.float32(2.0**30))
        idx_col = jnp.min(cand_f, axis=1, keepdims=True)
        idx_ref[...] = jax.lax.transpose(
            idx_col.astype(jnp.int32), (1, 0))


def _tc_argmin(x, codebook):
    batch, dim = x.shape
    n_codes = codebook.shape[0]
    n_steps = n_codes // K_BLOCK
    grid_kernel = functools.partial(_argmin_dist_kernel, n_steps=n_steps)
    return pl.pallas_call(
        grid_kernel,
        grid=(n_steps,),
        in_specs=[
            pl.BlockSpec((batch, dim), lambda k: (0, 0)),
            pl.BlockSpec((K_BLOCK, dim), lambda k: (k, 0)),
        ],
        out_specs=pl.BlockSpec((1, batch), lambda k: (0, 0)),
        out_shape=jax.ShapeDtypeStruct((1, batch), jnp.int32),
        scratch_shapes=[
            pltpu.VMEM((batch, 128), jnp.float32),
            pltpu.VMEM((batch, 128), jnp.float32),
            pltpu.VMEM((batch, dim), jnp.float32),
            pltpu.VMEM((batch, K_BLOCK), jnp.float32),
        ],
        compiler_params=pltpu.CompilerParams(
            dimension_semantics=("arbitrary",),
        ),
    )(x, codebook)


def _sc_gather(codebook, idx_row):
    n_codes, dim = codebook.shape
    batch = idx_row.shape[1]
    mesh = plsc.VectorSubcoreMesh(core_axis_name="core",
                                  subcore_axis_name="subcore")

    @functools.partial(
        pl.kernel,
        out_type=jax.ShapeDtypeStruct((batch, dim), jnp.float32),
        mesh=mesh,
    )
    def sc_kernel(cb_hbm, i_hbm, o_hbm):
        def body(i_vmem, o_vmem):
            pltpu.sync_copy(cb_hbm.at[i_vmem.at[0]], o_vmem)

        pltpu.emit_pipeline(
            body,
            grid=(batch // SC_WINDOW,),
            in_specs=[
                pl.BlockSpec((1, SC_WINDOW), lambda i: (0, i)),
            ],
            out_specs=[pl.BlockSpec((SC_WINDOW, dim), lambda i: (i, 0))],
            core_axis_name=("core", "subcore"),
            dimension_semantics=(pltpu.PARALLEL,),
        )(i_hbm, o_hbm)

    return sc_kernel(codebook, idx_row)


def kernel(previous_residual, codebook_embeddings):
    batch = previous_residual.shape[0]
    idx_row = _tc_argmin(previous_residual, codebook_embeddings)
    gathered = _sc_gather(codebook_embeddings, idx_row)
    next_residual = previous_residual - gathered
    return (idx_row.reshape(batch), next_residual)
